# baseline (speedup 1.0000x reference)
import functools
import math

import jax
import jax.numpy as jnp
from jax import lax
from jax.experimental import pallas as pl
from jax.experimental.pallas import tpu as pltpu
from jax.experimental.pallas import tpu_sc as plsc

BATCH = 4096
SEQ = 200
DIM = 64
LANES = 16
NUM_CORES = 2
NUM_SUBCORES = 16
NW = NUM_CORES * NUM_SUBCORES
TOK = BATCH * SEQ
TOK_PER_W = TOK // NW
CHUNK = 128
NCHUNK = TOK_PER_W // CHUNK
VPR = DIM // LANES
EXT = SEQ + CHUNK
OROWS = CHUNK * DIM // 128
SCALE = math.sqrt(DIM)


def _body(tok_hbm, lex2_hbm, pos2_hbm, out_hbm,
          idx_v, par_v, pos_v, gbuf0, gbuf1, obuf0, obuf1,
          gsem0, gsem1, osem0, osem1):
    wid = lax.axis_index("s") * NUM_CORES + lax.axis_index("c")
    row0 = wid * NCHUNK
    orow0 = wid * NCHUNK * OROWS

    pltpu.sync_copy(tok_hbm.at[pl.ds(row0, NCHUNK)], idx_v)
    pltpu.sync_copy(pos2_hbm.at[pl.ds(0, 104)], pos_v.at[pl.ds(0, 104)])
    pltpu.sync_copy(pos2_hbm.at[pl.ds(0, CHUNK // 2)],
                    pos_v.at[pl.ds(SEQ // 2, CHUNK // 2)])

    @plsc.parallel_loop(0, NCHUNK, unroll=2)
    def split(r):
        for k in range(CHUNK // LANES):
            sl = pl.ds(k * LANES, LANES)
            raw = idx_v[r, sl]
            par_v[r, sl] = (raw & 1) * DIM
            idx_v[r, sl] = raw >> 1

    gbufs = (gbuf0, gbuf1)
    obufs = (obuf0, obuf1)
    gsems = (gsem0, gsem1)
    osems = (osem0, osem1)

    def fire(g, b):
        pltpu.make_async_copy(
            lex2_hbm.at[idx_v.at[g]], gbufs[b], gsems[b]).start()

    def wait_gather(b):
        pltpu.make_async_copy(
            lex2_hbm.at[idx_v.at[0]], gbufs[b], gsems[b]).wait()

    def start_out(g, b):
        pltpu.make_async_copy(
            obufs[b], out_hbm.at[pl.ds(orow0 + g * OROWS, OROWS)],
            osems[b]).start()

    def wait_out(b):
        pltpu.make_async_copy(
            obufs[b], out_hbm.at[pl.ds(orow0, OROWS)], osems[b]).wait()

    def compute(g, b):
        gbuf = gbufs[b]
        obuf = obufs[b]
        s0 = lax.rem(g * CHUNK, SEQ)

        @plsc.parallel_loop(0, CHUNK // LANES, unroll=2)
        def group(gi):
            parv = par_v[g, pl.ds(gi * LANES, LANES)]
            for lane in range(LANES):
                off = parv[lane]
                i = gi * LANES + lane
                pv = (s0 + i) * VPR
                ov = i * VPR
                for k in range(VPR):
                    src = pl.ds(off + k * LANES, LANES)
                    p = pv + k
                    o = ov + k
                    val = (gbuf[i, src] * SCALE
                           + pos_v[p >> 3, pl.ds((p & 7) * LANES, LANES)])
                    obuf[o >> 3, pl.ds((o & 7) * LANES, LANES)] = val

    fire(0, 0)

    def pair(j, _):
        for b in range(2):
            g = 2 * j + b
            nxt = g + 1

            @pl.when(nxt < NCHUNK)
            def _():
                fire(nxt, 1 - b)

            wait_gather(b)

            @pl.when(j >= 1)
            def _():
                wait_out(b)

            compute(g, b)
            start_out(g, b)
        return _

    lax.fori_loop(0, NCHUNK // 2, pair, None)
    wait_out(0)
    wait_out(1)


@jax.jit
def _sc_embed(tok2, lex2, pos2):
    mesh = plsc.VectorSubcoreMesh(core_axis_name="c", subcore_axis_name="s")
    kern = functools.partial(
        pl.kernel,
        out_type=jax.ShapeDtypeStruct((TOK * DIM // 128, 128), jnp.float32),
        mesh=mesh,
        compiler_params=pltpu.CompilerParams(use_tc_tiling_on_sc=True),
        scratch_types=[
            pltpu.VMEM((NCHUNK, CHUNK), jnp.int32),
            pltpu.VMEM((NCHUNK, CHUNK), jnp.int32),
            pltpu.VMEM((EXT * DIM // 128, 128), jnp.float32),
            pltpu.VMEM((CHUNK, 128), jnp.float32),
            pltpu.VMEM((CHUNK, 128), jnp.float32),
            pltpu.VMEM((OROWS, 128), jnp.float32),
            pltpu.VMEM((OROWS, 128), jnp.float32),
            pltpu.SemaphoreType.DMA,
            pltpu.SemaphoreType.DMA,
            pltpu.SemaphoreType.DMA,
            pltpu.SemaphoreType.DMA,
        ],
    )(_body)
    return kern(tok2, lex2, pos2)


def kernel(token_indices, lexical_weight, positional_weight):
    b, s = token_indices.shape
    v, d = lexical_weight.shape
    m, _ = positional_weight.shape
    tok2 = token_indices.reshape(b * s // 128, 128)
    lex2 = lexical_weight.reshape(v // 2, 2 * d)
    pos2 = positional_weight.reshape(m * d // 128, 128)
    out = _sc_embed(tok2, lex2, pos2)
    return out.reshape(b, s, d)
